# baseline (speedup 1.0000x reference)
import functools

import jax
import jax.numpy as jnp
from jax import lax
from jax.experimental import pallas as pl
from jax.experimental.pallas import tpu as pltpu
from jax.experimental.pallas import tpu_sc as plsc

_N, _D, _H, _O, _B = 100000, 128, 128, 128, 1024
_BN = 2000
_SLICES = (44000, 42000)
_TAIL = 14000
_CH = 128
_NW = 32
_STRIPE = _B // 16
_NBUF = 4


def _mlp_body(x_ref, w1_ref, b1_ref, w2_ref, b2_ref, h_ref):
    x = x_ref[...]
    h = jnp.dot(x, w1_ref[...], preferred_element_type=jnp.float32)
    h = jnp.maximum(h + b1_ref[...], 0.0)
    h = jnp.dot(h, w2_ref[...], preferred_element_type=jnp.float32)
    h_ref[...] = jnp.maximum(h + b2_ref[...], 0.0)


def _mlp(x, W1, b1, W2, b2, base_blk, nb):
    return pl.pallas_call(
        _mlp_body,
        grid=(nb,),
        in_specs=[
            pl.BlockSpec((_BN, _D), lambda g: (g + base_blk, 0)),
            pl.BlockSpec((_D, _H), lambda g: (0, 0)),
            pl.BlockSpec((1, _H), lambda g: (0, 0)),
            pl.BlockSpec((_H, _H), lambda g: (0, 0)),
            pl.BlockSpec((1, _H), lambda g: (0, 0)),
        ],
        out_specs=pl.BlockSpec((_BN, _H), lambda g: (g, 0)),
        out_shape=jax.ShapeDtypeStruct((nb * _BN, _H), jnp.float32),
    )(x, W1, b1.reshape(1, _H), W2, b2.reshape(1, _H))


def _make_segsum_body(base_row, nrows, rem):
    nfull = nrows // _CH
    tmax = -(-nfull // _NW)
    ntail = nfull % _NW

    def body(h_hbm, ids_hbm, zeros_hbm, out_hbm,
             idx0, idx1, idx2, idx3, rows0, rows1, rows2, rows3,
             idx_r, rows_r, acc_sh,
             sl0, sl1, sl2, sl3, ss0, ss1, ss2, ss3):
        cid = lax.axis_index("c")
        sid = lax.axis_index("s")
        wid = sid * 2 + cid
        idx = (idx0, idx1, idx2, idx3)
        rows = (rows0, rows1, rows2, rows3)
        sem_l = (sl0, sl1, sl2, sl3)
        sem_s = (ss0, ss1, ss2, ss3)

        pltpu.sync_copy(zeros_hbm.at[pl.ds(sid * _STRIPE, _STRIPE)],
                        acc_sh.at[pl.ds(sid * _STRIPE, _STRIPE)])

        def valid(t):
            return t < tmax - 1 or ntail == 0

        def guard(t, fn):
            if valid(t):
                fn()
            else:
                pl.when(wid < ntail)(fn)

        def start_load(t, b):
            base = (wid + _NW * t) * _CH
            pltpu.async_copy(ids_hbm.at[pl.ds(base_row + base, _CH)],
                             idx[b], sem_l[b])
            pltpu.async_copy(h_hbm.at[pl.ds(base, _CH)], rows[b], sem_l[b])

        def issue_scatter(t, b):
            base = (wid + _NW * t) * _CH
            pltpu.make_async_copy(ids_hbm.at[pl.ds(base_row + base, _CH)],
                                  idx[b], sem_l[b]).wait()
            pltpu.make_async_copy(h_hbm.at[pl.ds(base, _CH)], rows[b],
                                  sem_l[b]).wait()
            pltpu.async_copy(rows[b], acc_sh.at[idx[b]], sem_s[b], add=True)

        def wait_scatter(t, b):
            pltpu.make_async_copy(rows[b], acc_sh.at[idx[b]],
                                  sem_s[b]).wait()

        guard(0, lambda: start_load(0, 0))
        plsc.subcore_barrier()

        for t in range(tmax):
            b = t % _NBUF
            if t >= _NBUF - 1:
                guard(t - _NBUF + 1,
                      functools.partial(wait_scatter, t - _NBUF + 1,
                                        (t - _NBUF + 1) % _NBUF))
            if t + 1 < tmax:
                guard(t + 1, functools.partial(start_load, t + 1,
                                               (t + 1) % _NBUF))
            guard(t, functools.partial(issue_scatter, t, b))
        for t in range(max(0, tmax - _NBUF + 1), tmax):
            guard(t, functools.partial(wait_scatter, t, t % _NBUF))

        if rem > 0:
            @pl.when(wid == _NW - 1)
            def _rem():
                base = nfull * _CH
                pltpu.sync_copy(ids_hbm.at[pl.ds(base_row + base, rem)],
                                idx_r)
                pltpu.sync_copy(h_hbm.at[pl.ds(base, rem)], rows_r)
                pltpu.sync_copy(rows_r, acc_sh.at[idx_r], add=True)

        plsc.subcore_barrier()
        out_base = cid * _B + sid * _STRIPE
        pltpu.sync_copy(acc_sh.at[pl.ds(sid * _STRIPE, _STRIPE)],
                        out_hbm.at[pl.ds(out_base, _STRIPE)])

    return body


def _segsum(h, ids, zeros, base_row, nrows):
    rem = nrows % _CH
    mesh = plsc.VectorSubcoreMesh(core_axis_name="c", subcore_axis_name="s")
    f = functools.partial(
        pl.kernel,
        mesh=mesh,
        out_type=jax.ShapeDtypeStruct((2 * _B, _H), jnp.float32),
        scratch_types=(
            [pltpu.VMEM((_CH,), jnp.int32)] * _NBUF
            + [pltpu.VMEM((_CH, _H), jnp.float32)] * _NBUF
            + [pltpu.VMEM((max(rem, 8),), jnp.int32),
               pltpu.VMEM((max(rem, 8), _H), jnp.float32),
               pltpu.VMEM_SHARED((_B, _H), jnp.float32)]
            + [pltpu.SemaphoreType.DMA] * (2 * _NBUF)
        ),
    )(_make_segsum_body(base_row, nrows, rem))
    return f(h, ids, zeros)


def _tail_body(ids_ref, x_ref, w1_ref, b1_ref, w2_ref, b2_ref,
               out_ref, acc_ref):
    g = pl.program_id(0)
    nb = pl.num_programs(0)

    @pl.when(g == 0)
    def _init():
        acc_ref[...] = jnp.zeros_like(acc_ref)

    x = x_ref[...]
    h = jnp.dot(x, w1_ref[...], preferred_element_type=jnp.float32)
    h = jnp.maximum(h + b1_ref[...], 0.0)
    h = jnp.dot(h, w2_ref[...], preferred_element_type=jnp.float32)
    h = jnp.maximum(h + b2_ref[...], 0.0)
    ids = ids_ref[0, 0, :]
    onehot = (jax.lax.broadcasted_iota(jnp.int32, (_BN, _B), 1)
              == ids[:, None]).astype(jnp.float32)
    acc_ref[...] += jax.lax.dot_general(
        onehot, h, (((0,), (0,)), ((), ())),
        preferred_element_type=jnp.float32)

    @pl.when(g == nb - 1)
    def _final():
        out_ref[...] = acc_ref[...]


def _tail_segsum(x, ids3, W1, b1, W2, b2, base_blk, nb):
    return pl.pallas_call(
        _tail_body,
        grid=(nb,),
        in_specs=[
            pl.BlockSpec((1, 1, _BN), lambda g: (g + base_blk, 0, 0)),
            pl.BlockSpec((_BN, _D), lambda g: (g + base_blk, 0)),
            pl.BlockSpec((_D, _H), lambda g: (0, 0)),
            pl.BlockSpec((1, _H), lambda g: (0, 0)),
            pl.BlockSpec((_H, _H), lambda g: (0, 0)),
            pl.BlockSpec((1, _H), lambda g: (0, 0)),
        ],
        out_specs=pl.BlockSpec((_B, _H), lambda g: (0, 0)),
        out_shape=jax.ShapeDtypeStruct((_B, _H), jnp.float32),
        scratch_shapes=[pltpu.VMEM((_B, _H), jnp.float32)],
    )(ids3, x, W1, b1.reshape(1, _H), W2, b2.reshape(1, _H))


def _out_body(p0_ref, p1_ref, pt_ref, wout_ref, bout_ref, out_ref):
    acc = (p0_ref[0:_B, :] + p0_ref[_B:2 * _B, :]
           + p1_ref[0:_B, :] + p1_ref[_B:2 * _B, :]
           + pt_ref[...])
    out_ref[...] = (jnp.dot(acc, wout_ref[...],
                            preferred_element_type=jnp.float32)
                    + bout_ref[...])


def _out_layer(partials, tail_partial, Wout, bout):
    pspec = pl.BlockSpec((2 * _B, _H), lambda: (0, 0))
    return pl.pallas_call(
        _out_body,
        in_specs=[pspec, pspec,
                  pl.BlockSpec((_B, _H), lambda: (0, 0)),
                  pl.BlockSpec((_H, _O), lambda: (0, 0)),
                  pl.BlockSpec((1, _O), lambda: (0, 0))],
        out_specs=pl.BlockSpec((_B, _O), lambda: (0, 0)),
        out_shape=jax.ShapeDtypeStruct((_B, _O), jnp.float32),
    )(*partials, tail_partial, Wout, bout.reshape(1, _O))


def kernel(node_features, batch_vector, W1, b1, W2, b2, Wout, bout):
    ids = batch_vector.astype(jnp.int32)
    ids3 = ids.reshape(_N // _BN, 1, _BN)
    zeros = jnp.zeros((_B, _H), jnp.float32)
    partials = []
    base = 0
    for nrows in _SLICES:
        h_s = _mlp(node_features, W1, b1, W2, b2, base // _BN, nrows // _BN)
        partials.append(_segsum(h_s, ids, zeros, base, nrows))
        base += nrows
    tail_partial = _tail_segsum(node_features, ids3, W1, b1, W2, b2,
                                base // _BN, _TAIL // _BN)
    return _out_layer(partials, tail_partial, Wout, bout)


# speedup vs baseline: 1.0047x; 1.0047x over previous
import functools

import numpy as np

import jax
import jax.numpy as jnp
from jax import lax
from jax.experimental import pallas as pl
from jax.experimental.pallas import tpu as pltpu
from jax.experimental.pallas import tpu_sc as plsc

_N, _D, _H, _O, _B = 100000, 128, 128, 128, 1024
_BN = 2000
_SLICES = (44000, 42000)
_TAIL = 14000
_CH = 128
_NW = 32
_STRIPE = _B // 16
_NBUF = 4

_ZEROS = np.zeros((_B, _H), np.float32)


def _mlp_body(x_ref, w1_ref, b1_ref, w2_ref, b2_ref, h_ref):
    x = x_ref[...]
    h = jnp.dot(x, w1_ref[...], preferred_element_type=jnp.float32)
    h = jnp.maximum(h + b1_ref[...], 0.0)
    h = jnp.dot(h, w2_ref[...], preferred_element_type=jnp.float32)
    h_ref[...] = jnp.maximum(h + b2_ref[...], 0.0)


def _mlp(x, W1, b1, W2, b2, base_blk, nb):
    return pl.pallas_call(
        _mlp_body,
        grid=(nb,),
        in_specs=[
            pl.BlockSpec((_BN, _D), lambda g: (g + base_blk, 0)),
            pl.BlockSpec((_D, _H), lambda g: (0, 0)),
            pl.BlockSpec((1, _H), lambda g: (0, 0)),
            pl.BlockSpec((_H, _H), lambda g: (0, 0)),
            pl.BlockSpec((1, _H), lambda g: (0, 0)),
        ],
        out_specs=pl.BlockSpec((_BN, _H), lambda g: (g, 0)),
        out_shape=jax.ShapeDtypeStruct((nb * _BN, _H), jnp.float32),
    )(x, W1, b1.reshape(1, _H), W2, b2.reshape(1, _H))


def _make_segsum_body(base_row, nrows, rem):
    nfull = nrows // _CH
    tmax = -(-nfull // _NW)
    ntail = nfull % _NW

    def body(h_hbm, ids_hbm, zeros_hbm, out_hbm,
             idx0, idx1, idx2, idx3, rows0, rows1, rows2, rows3,
             idx_r, rows_r, acc_sh,
             sl0, sl1, sl2, sl3, ss0, ss1, ss2, ss3):
        cid = lax.axis_index("c")
        sid = lax.axis_index("s")
        wid = sid * 2 + cid
        idx = (idx0, idx1, idx2, idx3)
        rows = (rows0, rows1, rows2, rows3)
        sem_l = (sl0, sl1, sl2, sl3)
        sem_s = (ss0, ss1, ss2, ss3)

        pltpu.sync_copy(zeros_hbm.at[pl.ds(sid * _STRIPE, _STRIPE)],
                        acc_sh.at[pl.ds(sid * _STRIPE, _STRIPE)])

        def valid(t):
            return t < tmax - 1 or ntail == 0

        def guard(t, fn):
            if valid(t):
                fn()
            else:
                pl.when(wid < ntail)(fn)

        def start_load(t, b):
            base = (wid + _NW * t) * _CH
            pltpu.async_copy(ids_hbm.at[pl.ds(base_row + base, _CH)],
                             idx[b], sem_l[b])
            pltpu.async_copy(h_hbm.at[pl.ds(base, _CH)], rows[b], sem_l[b])

        def issue_scatter(t, b):
            base = (wid + _NW * t) * _CH
            pltpu.make_async_copy(ids_hbm.at[pl.ds(base_row + base, _CH)],
                                  idx[b], sem_l[b]).wait()
            pltpu.make_async_copy(h_hbm.at[pl.ds(base, _CH)], rows[b],
                                  sem_l[b]).wait()
            pltpu.async_copy(rows[b], acc_sh.at[idx[b]], sem_s[b], add=True)

        def wait_scatter(t, b):
            pltpu.make_async_copy(rows[b], acc_sh.at[idx[b]],
                                  sem_s[b]).wait()

        guard(0, lambda: start_load(0, 0))
        plsc.subcore_barrier()

        for t in range(tmax):
            b = t % _NBUF
            if t >= _NBUF - 1:
                guard(t - _NBUF + 1,
                      functools.partial(wait_scatter, t - _NBUF + 1,
                                        (t - _NBUF + 1) % _NBUF))
            if t + 1 < tmax:
                guard(t + 1, functools.partial(start_load, t + 1,
                                               (t + 1) % _NBUF))
            guard(t, functools.partial(issue_scatter, t, b))
        for t in range(max(0, tmax - _NBUF + 1), tmax):
            guard(t, functools.partial(wait_scatter, t, t % _NBUF))

        if rem > 0:
            @pl.when(wid == _NW - 1)
            def _rem():
                base = nfull * _CH
                pltpu.sync_copy(ids_hbm.at[pl.ds(base_row + base, rem)],
                                idx_r)
                pltpu.sync_copy(h_hbm.at[pl.ds(base, rem)], rows_r)
                pltpu.sync_copy(rows_r, acc_sh.at[idx_r], add=True)

        plsc.subcore_barrier()
        out_base = cid * _B + sid * _STRIPE
        pltpu.sync_copy(acc_sh.at[pl.ds(sid * _STRIPE, _STRIPE)],
                        out_hbm.at[pl.ds(out_base, _STRIPE)])

    return body


def _segsum(h, ids, zeros, base_row, nrows):
    rem = nrows % _CH
    mesh = plsc.VectorSubcoreMesh(core_axis_name="c", subcore_axis_name="s")
    f = functools.partial(
        pl.kernel,
        mesh=mesh,
        out_type=jax.ShapeDtypeStruct((2 * _B, _H), jnp.float32),
        scratch_types=(
            [pltpu.VMEM((_CH,), jnp.int32)] * _NBUF
            + [pltpu.VMEM((_CH, _H), jnp.float32)] * _NBUF
            + [pltpu.VMEM((max(rem, 8),), jnp.int32),
               pltpu.VMEM((max(rem, 8), _H), jnp.float32),
               pltpu.VMEM_SHARED((_B, _H), jnp.float32)]
            + [pltpu.SemaphoreType.DMA] * (2 * _NBUF)
        ),
    )(_make_segsum_body(base_row, nrows, rem))
    return f(h, ids, zeros)


def _tail_body(ids_ref, x_ref, w1_ref, b1_ref, w2_ref, b2_ref,
               out_ref, acc_ref):
    g = pl.program_id(0)
    nb = pl.num_programs(0)

    @pl.when(g == 0)
    def _init():
        acc_ref[...] = jnp.zeros_like(acc_ref)

    x = x_ref[...]
    h = jnp.dot(x, w1_ref[...], preferred_element_type=jnp.float32)
    h = jnp.maximum(h + b1_ref[...], 0.0)
    h = jnp.dot(h, w2_ref[...], preferred_element_type=jnp.float32)
    h = jnp.maximum(h + b2_ref[...], 0.0)
    ids = ids_ref[0, 0, :]
    onehot = (jax.lax.broadcasted_iota(jnp.int32, (_BN, _B), 1)
              == ids[:, None]).astype(jnp.float32)
    acc_ref[...] += jax.lax.dot_general(
        onehot, h, (((0,), (0,)), ((), ())),
        preferred_element_type=jnp.float32)

    @pl.when(g == nb - 1)
    def _final():
        out_ref[...] = acc_ref[...]


def _tail_segsum(x, ids3, W1, b1, W2, b2, base_blk, nb):
    return pl.pallas_call(
        _tail_body,
        grid=(nb,),
        in_specs=[
            pl.BlockSpec((1, 1, _BN), lambda g: (g + base_blk, 0, 0)),
            pl.BlockSpec((_BN, _D), lambda g: (g + base_blk, 0)),
            pl.BlockSpec((_D, _H), lambda g: (0, 0)),
            pl.BlockSpec((1, _H), lambda g: (0, 0)),
            pl.BlockSpec((_H, _H), lambda g: (0, 0)),
            pl.BlockSpec((1, _H), lambda g: (0, 0)),
        ],
        out_specs=pl.BlockSpec((_B, _H), lambda g: (0, 0)),
        out_shape=jax.ShapeDtypeStruct((_B, _H), jnp.float32),
        scratch_shapes=[pltpu.VMEM((_B, _H), jnp.float32)],
    )(ids3, x, W1, b1.reshape(1, _H), W2, b2.reshape(1, _H))


def _out_body(p0_ref, p1_ref, pt_ref, wout_ref, bout_ref, out_ref):
    acc = (p0_ref[0:_B, :] + p0_ref[_B:2 * _B, :]
           + p1_ref[0:_B, :] + p1_ref[_B:2 * _B, :]
           + pt_ref[...])
    out_ref[...] = (jnp.dot(acc, wout_ref[...],
                            preferred_element_type=jnp.float32)
                    + bout_ref[...])


def _out_layer(partials, tail_partial, Wout, bout):
    pspec = pl.BlockSpec((2 * _B, _H), lambda: (0, 0))
    return pl.pallas_call(
        _out_body,
        in_specs=[pspec, pspec,
                  pl.BlockSpec((_B, _H), lambda: (0, 0)),
                  pl.BlockSpec((_H, _O), lambda: (0, 0)),
                  pl.BlockSpec((1, _O), lambda: (0, 0))],
        out_specs=pl.BlockSpec((_B, _O), lambda: (0, 0)),
        out_shape=jax.ShapeDtypeStruct((_B, _O), jnp.float32),
    )(*partials, tail_partial, Wout, bout.reshape(1, _O))


def kernel(node_features, batch_vector, W1, b1, W2, b2, Wout, bout):
    ids = batch_vector.astype(jnp.int32)
    ids3 = ids.reshape(_N // _BN, 1, _BN)
    zeros = jnp.asarray(_ZEROS)
    partials = []
    base = 0
    for nrows in _SLICES:
        h_s = _mlp(node_features, W1, b1, W2, b2, base // _BN, nrows // _BN)
        partials.append(_segsum(h_s, ids, zeros, base, nrows))
        base += nrows
    tail_partial = _tail_segsum(node_features, ids3, W1, b1, W2, b2,
                                base // _BN, _TAIL // _BN)
    return _out_layer(partials, tail_partial, Wout, bout)
